# baseline (speedup 1.0000x reference)
import functools

import jax
import jax.numpy as jnp
from jax import lax
from jax.experimental import pallas as pl
from jax.experimental.pallas import tpu as pltpu
from jax.experimental.pallas import tpu_sc as plsc

B = 128
V = 100000
N = B * V
HIST = 200
PRANGE = 100
L = 16
NIDX = 112
NKEY = B * NIDX
NC, NS = 2, 16
NW = NC * NS
CHB = 14
CH = 1 << CHB
NFULL = N // CH
TAILW = N - NFULL * CH
NT = 24
NBUF = 4
CAP = NKEY + L


def _filter_chunk(buf, kept, cidx, cvals, pen, c, kept_n, goff):
    lanes = lax.iota(jnp.int32, L)
    nv = (kept_n + L - 1) >> 4

    def body(i, off):
        kv = kept[pl.ds(i * L, L)]
        m = ((i * L + lanes) < kept_n) & ((kv >> CHB) == c)

        def hit(off):
            local = jnp.where(m, kv & (CH - 1), 0)
            g = plsc.load_gather(buf, [local], mask=m)
            plsc.store_compressed(cidx.at[pl.ds(off, L)], kv, mask=m)
            plsc.store_compressed(cvals.at[pl.ds(off, L)], g * pen, mask=m)
            return off + jnp.max(plsc.all_reduce_population_count(m))

        return lax.cond(jnp.any(m), hit, lambda o: o, off)

    return lax.fori_loop(0, nv, body, goff)


def _body(x_hbm, keys_hbm, pen_hbm, o_hbm, *refs):
    bufs = refs[:NBUF]
    keysv, kept, cidx, cvals, penv, padidx, padval = refs[NBUF:NBUF + 7]
    insems = refs[NBUF + 7: 2 * NBUF + 7]
    outsems = refs[2 * NBUF + 7: 3 * NBUF + 7]
    scatsem = refs[3 * NBUF + 7]

    w = lax.axis_index("s") * NC + lax.axis_index("c")
    lanes = lax.iota(jnp.int32, L)

    def chunk_off(t):
        return (w + 32 * t) * CH

    ins = [None] * NT
    for j in range(2):
        ins[j] = pltpu.async_copy(x_hbm.at[pl.ds(chunk_off(j), CH)],
                                  bufs[j], insems[j])
    pltpu.sync_copy(pen_hbm, penv)
    pltpu.sync_copy(keys_hbm, keysv)
    pen = penv[...]

    def scan_body(i, off):
        iv = keysv[pl.ds(i * L, L)]
        m = ((iv >> CHB) & (NW - 1)) == w

        def hit(off):
            plsc.store_compressed(kept.at[pl.ds(off, L)], iv, mask=m)
            return off + jnp.max(plsc.all_reduce_population_count(m))

        return lax.cond(jnp.any(m), hit, lambda o: o, off)

    kept_n = lax.fori_loop(0, NKEY // L, scan_body, 0)

    def flush(fr, to):
        def fbody(r, carry):
            pltpu.async_copy(cvals.at[pl.ds(r * L, L)],
                             o_hbm.at[cidx.at[pl.ds(r * L, L)]], scatsem)
            return carry
        lax.fori_loop(fr, to, fbody, 0)
        return to

    outs = [None] * NT
    goff = jnp.int32(0)
    goffs = []
    fvreg = jnp.int32(0)
    for t in range(NT):
        b = t % NBUF
        ins[t].wait()
        nxt = t + 2
        if nxt < NT:
            ins[nxt] = pltpu.async_copy(
                x_hbm.at[pl.ds(chunk_off(nxt), CH)], bufs[nxt % NBUF],
                insems[nxt % NBUF])
        goff = _filter_chunk(bufs[b], kept, cidx, cvals, pen,
                             w + 32 * t, kept_n, goff)
        goffs.append(goff)
        outs[t] = pltpu.async_copy(bufs[b], o_hbm.at[pl.ds(chunk_off(t), CH)],
                                   outsems[b])
        if t >= 1:
            outs[t - 1].wait()
            fvreg = flush(fvreg, goffs[t - 1] >> 4)
    outs[NT - 1].wait()

    ec = w + 32 * NT
    eoff = ec * CH

    @pl.when(w < 13)
    def _efull():
        pltpu.sync_copy(x_hbm.at[pl.ds(eoff, CH)], bufs[0])

    @pl.when(w == 13)
    def _etail_in():
        pltpu.sync_copy(x_hbm.at[pl.ds(NFULL * CH, TAILW)],
                        bufs[0].at[pl.ds(0, TAILW)])

    goff = _filter_chunk(bufs[0], kept, cidx, cvals, pen, ec, kept_n, goff)

    @pl.when(w < 13)
    def _efull_out():
        pltpu.sync_copy(bufs[0], o_hbm.at[pl.ds(eoff, CH)])

    @pl.when(w == 13)
    def _etail_out():
        pltpu.sync_copy(bufs[0].at[pl.ds(0, TAILW)],
                        o_hbm.at[pl.ds(NFULL * CH, TAILW)])

    fvreg = flush(fvreg, goff >> 4)
    rem = goff & (L - 1)
    base = goff - rem

    @pl.when(rem > 0)
    def _remainder():
        kv = cidx[pl.ds(base, L)]
        vv = cvals[pl.ds(base, L)]
        zero16 = jnp.zeros((L,), jnp.int32)
        k0 = kv.at[zero16].get(mode="promise_in_bounds")
        v0 = vv.at[zero16].get(mode="promise_in_bounds")
        mfix = lanes < rem
        padidx[...] = jnp.where(mfix, kv, k0)
        padval[...] = jnp.where(mfix, vv, v0)
        pl.delay(100)
        pltpu.async_copy(padval, o_hbm.at[padidx], scatsem)

    nscat = fvreg + jnp.where(rem > 0, 1, 0)

    def dbody(r, carry):
        pltpu.make_async_copy(cvals.at[pl.ds(0, L)],
                              o_hbm.at[cidx.at[pl.ds(0, L)]], scatsem).wait()
        return carry
    lax.fori_loop(0, nscat, dbody, 0)


def kernel(logits, save_id, penalty_value, penality_range):
    del penality_range
    tgt = save_id[:, HIST - PRANGE:].astype(jnp.int32)
    keys = tgt * B + jnp.arange(B, dtype=jnp.int32)[:, None]
    keys = jnp.concatenate([keys, keys[:, : NIDX - PRANGE]], axis=1)
    keys = keys.reshape(NKEY)
    pen16 = jnp.broadcast_to(penalty_value.astype(jnp.float32), (L,))
    x = logits.T.reshape(N)

    run = functools.partial(
        pl.kernel,
        out_type=jax.ShapeDtypeStruct((N,), jnp.float32),
        mesh=plsc.VectorSubcoreMesh(
            core_axis_name="c", subcore_axis_name="s",
            num_cores=NC, num_subcores=NS,
        ),
        scratch_types=(
            [pltpu.VMEM((CH,), jnp.float32)] * NBUF
            + [pltpu.VMEM((NKEY,), jnp.int32),
               pltpu.VMEM((CAP,), jnp.int32),
               pltpu.VMEM((CAP,), jnp.int32),
               pltpu.VMEM((CAP,), jnp.float32),
               pltpu.VMEM((L,), jnp.float32),
               pltpu.VMEM((L,), jnp.int32),
               pltpu.VMEM((L,), jnp.float32)]
            + [pltpu.SemaphoreType.DMA] * (2 * NBUF + 1)
        ),
        compiler_params=pltpu.CompilerParams(needs_layout_passes=False),
    )(_body)
    return run(x, keys, pen16).reshape(V, B).T


# speedup vs baseline: 1.0639x; 1.0639x over previous
import functools

import jax
import jax.numpy as jnp
from jax import lax
from jax.experimental import pallas as pl
from jax.experimental.pallas import tpu as pltpu
from jax.experimental.pallas import tpu_sc as plsc

B = 128
V = 100000
N = B * V
HIST = 200
PRANGE = 100
L = 16
NIDX = 112
NKEY = B * NIDX
NC, NS = 2, 16
NW = NC * NS
CHB = 14
CH = 1 << CHB
NFULL = N // CH
TAILW = N - NFULL * CH
NT = 24
NBUF = 4
CAP = NKEY + L


def _filter_chunk(buf, kept, cidx, cvals, pen, c, kept_n, goff):
    lanes = lax.iota(jnp.int32, L)
    nv = (kept_n + L - 1) >> 4

    def body(i, off):
        kv = kept[pl.ds(i * L, L)]
        m = ((i * L + lanes) < kept_n) & ((kv >> CHB) == c)

        def hit(off):
            local = jnp.where(m, kv & (CH - 1), 0)
            g = plsc.load_gather(buf, [local], mask=m)
            plsc.store_compressed(cidx.at[pl.ds(off, L)], kv, mask=m)
            plsc.store_compressed(cvals.at[pl.ds(off, L)], g * pen, mask=m)
            return off + jnp.max(plsc.all_reduce_population_count(m))

        return lax.cond(jnp.any(m), hit, lambda o: o, off)

    return lax.fori_loop(0, nv, body, goff)


def _apply_chunk(buf, cidx, cvals, lo, hi):
    lanes = lax.iota(jnp.int32, L)

    def body(r, carry):
        kv = cidx[pl.ds(r * L, L)]
        vv = cvals[pl.ds(r * L, L)]
        pos = r * L + lanes
        m = (pos >= lo) & (pos < hi)
        local = jnp.where(m, kv & (CH - 1), 0)
        plsc.store_scatter(buf, [local], vv, mask=m)
        return carry

    lax.fori_loop(lo >> 4, (hi + L - 1) >> 4, body, 0)


def _body(x_hbm, keys_hbm, pen_hbm, o_hbm, *refs):
    bufs = refs[:NBUF]
    keysv, kept, cidx, cvals, penv = refs[NBUF:NBUF + 5]
    insems = refs[NBUF + 5: 2 * NBUF + 5]
    outsems = refs[2 * NBUF + 5: 3 * NBUF + 5]

    w = lax.axis_index("s") * NC + lax.axis_index("c")
    lanes = lax.iota(jnp.int32, L)

    def chunk_off(t):
        return (w + 32 * t) * CH

    ins = [None] * NT
    for j in range(2):
        ins[j] = pltpu.async_copy(x_hbm.at[pl.ds(chunk_off(j), CH)],
                                  bufs[j], insems[j])
    pltpu.sync_copy(pen_hbm, penv)
    pltpu.sync_copy(keys_hbm, keysv)
    pen = penv[...]

    def scan_body(i, off):
        iv = keysv[pl.ds(i * L, L)]
        m = ((iv >> CHB) & (NW - 1)) == w

        def hit(off):
            plsc.store_compressed(kept.at[pl.ds(off, L)], iv, mask=m)
            return off + jnp.max(plsc.all_reduce_population_count(m))

        return lax.cond(jnp.any(m), hit, lambda o: o, off)

    kept_n = lax.fori_loop(0, NKEY // L, scan_body, 0)

    outs = [None] * NT
    goff = jnp.int32(0)
    for t in range(NT):
        b = t % NBUF
        ins[t].wait()
        nxt = t + 2
        if nxt < NT:
            ins[nxt] = pltpu.async_copy(
                x_hbm.at[pl.ds(chunk_off(nxt), CH)], bufs[nxt % NBUF],
                insems[nxt % NBUF])
        prev = goff
        goff = _filter_chunk(bufs[b], kept, cidx, cvals, pen,
                             w + 32 * t, kept_n, goff)
        _apply_chunk(bufs[b], cidx, cvals, prev, goff)
        outs[t] = pltpu.async_copy(bufs[b], o_hbm.at[pl.ds(chunk_off(t), CH)],
                                   outsems[b])
        if t >= 1:
            outs[t - 1].wait()
    outs[NT - 1].wait()

    ec = w + 32 * NT
    eoff = ec * CH

    @pl.when(w < 13)
    def _efull():
        pltpu.sync_copy(x_hbm.at[pl.ds(eoff, CH)], bufs[0])

    @pl.when(w == 13)
    def _etail_in():
        pltpu.sync_copy(x_hbm.at[pl.ds(NFULL * CH, TAILW)],
                        bufs[0].at[pl.ds(0, TAILW)])

    prev = goff
    goff = _filter_chunk(bufs[0], kept, cidx, cvals, pen, ec, kept_n, goff)
    _apply_chunk(bufs[0], cidx, cvals, prev, goff)

    @pl.when(w < 13)
    def _efull_out():
        pltpu.sync_copy(bufs[0], o_hbm.at[pl.ds(eoff, CH)])

    @pl.when(w == 13)
    def _etail_out():
        pltpu.sync_copy(bufs[0].at[pl.ds(0, TAILW)],
                        o_hbm.at[pl.ds(NFULL * CH, TAILW)])


def kernel(logits, save_id, penalty_value, penality_range):
    del penality_range
    tgt = save_id[:, HIST - PRANGE:].astype(jnp.int32)
    keys = tgt * B + jnp.arange(B, dtype=jnp.int32)[:, None]
    keys = jnp.concatenate([keys, keys[:, : NIDX - PRANGE]], axis=1)
    keys = keys.reshape(NKEY)
    pen16 = jnp.broadcast_to(penalty_value.astype(jnp.float32), (L,))
    x = logits.T.reshape(N)

    run = functools.partial(
        pl.kernel,
        out_type=jax.ShapeDtypeStruct((N,), jnp.float32),
        mesh=plsc.VectorSubcoreMesh(
            core_axis_name="c", subcore_axis_name="s",
            num_cores=NC, num_subcores=NS,
        ),
        scratch_types=(
            [pltpu.VMEM((CH,), jnp.float32)] * NBUF
            + [pltpu.VMEM((NKEY,), jnp.int32),
               pltpu.VMEM((CAP,), jnp.int32),
               pltpu.VMEM((CAP,), jnp.int32),
               pltpu.VMEM((CAP,), jnp.float32),
               pltpu.VMEM((L,), jnp.float32)]
            + [pltpu.SemaphoreType.DMA] * (2 * NBUF)
        ),
        compiler_params=pltpu.CompilerParams(needs_layout_passes=False),
    )(_body)
    return run(x, keys, pen16).reshape(V, B).T


# speedup vs baseline: 1.0658x; 1.0018x over previous
import functools

import jax
import jax.numpy as jnp
from jax import lax
from jax.experimental import pallas as pl
from jax.experimental.pallas import tpu as pltpu
from jax.experimental.pallas import tpu_sc as plsc

B = 128
V = 100000
N = B * V
HIST = 200
PRANGE = 100
L = 16
NIDX = 112
NKEY = B * NIDX
NC, NS = 2, 16
NW = NC * NS
CHB = 14
CH = 1 << CHB
NFULL = N // CH
TAILW = N - NFULL * CH
NT = 24
NBUF = 5
CAP = NKEY + L


def _filter_chunk(buf, kept, cidx, cvals, pen, c, kept_n, goff):
    lanes = lax.iota(jnp.int32, L)
    nv = (kept_n + L - 1) >> 4

    def body(i, off):
        kv = kept[pl.ds(i * L, L)]
        m = ((i * L + lanes) < kept_n) & ((kv >> CHB) == c)

        def hit(off):
            local = jnp.where(m, kv & (CH - 1), 0)
            g = plsc.load_gather(buf, [local], mask=m)
            plsc.store_compressed(cidx.at[pl.ds(off, L)], kv, mask=m)
            plsc.store_compressed(cvals.at[pl.ds(off, L)], g * pen, mask=m)
            return off + jnp.max(plsc.all_reduce_population_count(m))

        return lax.cond(jnp.any(m), hit, lambda o: o, off)

    return lax.fori_loop(0, nv, body, goff)


def _apply_chunk(buf, cidx, cvals, lo, hi):
    lanes = lax.iota(jnp.int32, L)

    def body(r, carry):
        kv = cidx[pl.ds(r * L, L)]
        vv = cvals[pl.ds(r * L, L)]
        pos = r * L + lanes
        m = (pos >= lo) & (pos < hi)
        local = jnp.where(m, kv & (CH - 1), 0)
        plsc.store_scatter(buf, [local], vv, mask=m)
        return carry

    lax.fori_loop(lo >> 4, (hi + L - 1) >> 4, body, 0)


def _body(x_hbm, keys_hbm, pen_hbm, o_hbm, *refs):
    bufs = refs[:NBUF]
    keysv, cidx, cvals, penv = refs[NBUF:NBUF + 4]
    insems = refs[NBUF + 4: 2 * NBUF + 4]
    outsems = refs[2 * NBUF + 4: 3 * NBUF + 4]

    w = lax.axis_index("s") * NC + lax.axis_index("c")
    lanes = lax.iota(jnp.int32, L)

    def chunk_off(t):
        return (w + 32 * t) * CH

    ins = [None] * NT
    for j in range(3):
        ins[j] = pltpu.async_copy(x_hbm.at[pl.ds(chunk_off(j), CH)],
                                  bufs[j], insems[j])
    pltpu.sync_copy(pen_hbm, penv)
    pltpu.sync_copy(keys_hbm, keysv.at[pl.ds(0, NKEY)])
    pen = penv[...]

    def scan_body(i, off):
        iv = keysv[pl.ds(i * L, L)]
        m = ((iv >> CHB) & (NW - 1)) == w

        def hit(off):
            plsc.store_compressed(keysv.at[pl.ds(off, L)], iv, mask=m)
            return off + jnp.max(plsc.all_reduce_population_count(m))

        return lax.cond(jnp.any(m), hit, lambda o: o, off)

    kept_n = lax.fori_loop(0, NKEY // L, scan_body, 0)

    outs = [None] * NT
    goff = jnp.int32(0)
    for t in range(NT):
        b = t % NBUF
        ins[t].wait()
        nxt = t + 3
        if nxt < NT:
            ins[nxt] = pltpu.async_copy(
                x_hbm.at[pl.ds(chunk_off(nxt), CH)], bufs[nxt % NBUF],
                insems[nxt % NBUF])
        prev = goff
        goff = _filter_chunk(bufs[b], keysv, cidx, cvals, pen,
                             w + 32 * t, kept_n, goff)
        _apply_chunk(bufs[b], cidx, cvals, prev, goff)
        outs[t] = pltpu.async_copy(bufs[b], o_hbm.at[pl.ds(chunk_off(t), CH)],
                                   outsems[b])
        if t >= 1:
            outs[t - 1].wait()
    outs[NT - 1].wait()

    ec = w + 32 * NT
    eoff = ec * CH

    @pl.when(w < 13)
    def _efull():
        pltpu.sync_copy(x_hbm.at[pl.ds(eoff, CH)], bufs[0])

    @pl.when(w == 13)
    def _etail_in():
        pltpu.sync_copy(x_hbm.at[pl.ds(NFULL * CH, TAILW)],
                        bufs[0].at[pl.ds(0, TAILW)])

    prev = goff
    goff = _filter_chunk(bufs[0], keysv, cidx, cvals, pen, ec, kept_n, goff)
    _apply_chunk(bufs[0], cidx, cvals, prev, goff)

    @pl.when(w < 13)
    def _efull_out():
        pltpu.sync_copy(bufs[0], o_hbm.at[pl.ds(eoff, CH)])

    @pl.when(w == 13)
    def _etail_out():
        pltpu.sync_copy(bufs[0].at[pl.ds(0, TAILW)],
                        o_hbm.at[pl.ds(NFULL * CH, TAILW)])


def kernel(logits, save_id, penalty_value, penality_range):
    del penality_range
    tgt = save_id[:, HIST - PRANGE:].astype(jnp.int32)
    keys = tgt * B + jnp.arange(B, dtype=jnp.int32)[:, None]
    keys = jnp.concatenate([keys, keys[:, : NIDX - PRANGE]], axis=1)
    keys = keys.reshape(NKEY)
    pen16 = jnp.broadcast_to(penalty_value.astype(jnp.float32), (L,))
    x = logits.T.reshape(N)

    run = functools.partial(
        pl.kernel,
        out_type=jax.ShapeDtypeStruct((N,), jnp.float32),
        mesh=plsc.VectorSubcoreMesh(
            core_axis_name="c", subcore_axis_name="s",
            num_cores=NC, num_subcores=NS,
        ),
        scratch_types=(
            [pltpu.VMEM((CH,), jnp.float32)] * NBUF
            + [pltpu.VMEM((CAP,), jnp.int32),
               pltpu.VMEM((CAP,), jnp.int32),
               pltpu.VMEM((CAP,), jnp.float32),
               pltpu.VMEM((L,), jnp.float32)]
            + [pltpu.SemaphoreType.DMA] * (2 * NBUF)
        ),
        compiler_params=pltpu.CompilerParams(needs_layout_passes=False),
    )(_body)
    return run(x, keys, pen16).reshape(V, B).T
